# Initial kernel scaffold; baseline (speedup 1.0000x reference)
import jax
import jax.numpy as jnp
from jax.experimental import pallas as pl


def kernel(z_src, z_dst, edge_label_index):
    raise NotImplementedError("write your pallas kernel here")


# speedup vs baseline: 2.3198x; 2.3198x over previous
import functools

import jax
import jax.numpy as jnp
from jax import lax
from jax.experimental import pallas as pl
from jax.experimental.pallas import tpu as pltpu
from jax.experimental.pallas import tpu_sc as plsc

N_NODES = 10000
D = 128
E = 320000
NC = 2
NS = 16
NW = NC * NS
EPW = E // NW
CHUNK = 80
NCHUNK = EPW // CHUNK
GROUPS = CHUNK // 16


def _body(zs_hbm, zd_hbm, sidx_hbm, didx_hbm, out_hbm,
          sidx_v, didx_v, zs_rows, zd_rows, out_v, trans_v, sem_s, sem_d):
    wid = lax.axis_index("s") * NC + lax.axis_index("c")
    base = wid * EPW
    lane16 = lax.iota(jnp.int32, 16) * 16

    def chunk_body(i, _):
        cbase = base + i * CHUNK
        pltpu.sync_copy(sidx_hbm.at[pl.ds(cbase, CHUNK)], sidx_v)
        pltpu.sync_copy(didx_hbm.at[pl.ds(cbase, CHUNK)], didx_v)
        cp_s = pltpu.async_copy(zs_hbm.at[sidx_v], zs_rows, sem_s)
        cp_d = pltpu.async_copy(zd_hbm.at[didx_v], zd_rows, sem_d)
        cp_s.wait()
        cp_d.wait()

        for g in range(GROUPS):
            for e in range(16):
                row = g * 16 + e
                parts = []
                for j in range(D // 16):
                    vs = zs_rows[row, pl.ds(j * 16, 16)]
                    vd = zd_rows[row, pl.ds(j * 16, 16)]
                    parts.append(vs * vd)
                while len(parts) > 1:
                    parts = [a + b for a, b in
                             zip(parts[::2], parts[1::2])]
                plsc.store_scatter(trans_v, [lane16 + e], parts[0])
            colsums = [trans_v[pl.ds(l * 16, 16)] for l in range(16)]
            while len(colsums) > 1:
                colsums = [a + b for a, b in
                           zip(colsums[::2], colsums[1::2])]
            out_v[pl.ds(g * 16, 16)] = colsums[0]

        pltpu.sync_copy(out_v, out_hbm.at[pl.ds(cbase, CHUNK)])
        return ()

    lax.fori_loop(0, NCHUNK, chunk_body, ())


@jax.jit
def _run(z_src, z_dst, src_idx, dst_idx):
    mesh = plsc.VectorSubcoreMesh(core_axis_name="c", subcore_axis_name="s")
    f = pl.kernel(
        _body,
        out_type=jax.ShapeDtypeStruct((E,), jnp.float32),
        mesh=mesh,
        compiler_params=pltpu.CompilerParams(needs_layout_passes=False),
        scratch_types=[
            pltpu.VMEM((CHUNK,), jnp.int32),
            pltpu.VMEM((CHUNK,), jnp.int32),
            pltpu.VMEM((CHUNK, D), jnp.float32),
            pltpu.VMEM((CHUNK, D), jnp.float32),
            pltpu.VMEM((CHUNK,), jnp.float32),
            pltpu.VMEM((256,), jnp.float32),
            pltpu.SemaphoreType.DMA,
            pltpu.SemaphoreType.DMA,
        ],
    )
    return f(z_src, z_dst, src_idx, dst_idx)


def kernel(z_src, z_dst, edge_label_index):
    return _run(z_src, z_dst, edge_label_index[0], edge_label_index[1])


# speedup vs baseline: 3.4674x; 1.4947x over previous
import jax
import jax.numpy as jnp
from jax import lax
from jax.experimental import pallas as pl
from jax.experimental.pallas import tpu as pltpu
from jax.experimental.pallas import tpu_sc as plsc

N_NODES = 10000
D = 128
E = 320000
NC = 2
NS = 16
NW = NC * NS
EPW = E // NW
CHUNK = 80
NCHUNK = EPW // CHUNK
GROUPS = CHUNK // 16


def _body(zs_hbm, zd_hbm, sidx_hbm, didx_hbm, out_hbm,
          sidx_v, didx_v, rows_s0, rows_d0, rows_s1, rows_d1, out_v,
          trans_v, sem_s0, sem_d0, sem_s1, sem_d1):
    wid = lax.axis_index("s") * NC + lax.axis_index("c")
    base = wid * EPW
    lane16 = lax.iota(jnp.int32, 16) * 16

    pltpu.sync_copy(sidx_hbm.at[pl.ds(base, EPW)], sidx_v)
    pltpu.sync_copy(didx_hbm.at[pl.ds(base, EPW)], didx_v)

    bufs = ((rows_s0, rows_d0, sem_s0, sem_d0),
            (rows_s1, rows_d1, sem_s1, sem_d1))

    def start_gather(c, b):
        rs, rd, ss, sd = bufs[b]
        pltpu.async_copy(zs_hbm.at[sidx_v.at[pl.ds(c * CHUNK, CHUNK)]], rs, ss)
        pltpu.async_copy(zd_hbm.at[didx_v.at[pl.ds(c * CHUNK, CHUNK)]], rd, sd)

    def wait_gather(c, b):
        rs, rd, ss, sd = bufs[b]
        pltpu.make_async_copy(
            zs_hbm.at[sidx_v.at[pl.ds(c * CHUNK, CHUNK)]], rs, ss).wait()
        pltpu.make_async_copy(
            zd_hbm.at[didx_v.at[pl.ds(c * CHUNK, CHUNK)]], rd, sd).wait()

    def compute(c, b):
        rs, rd, _, _ = bufs[b]
        obase = c * CHUNK
        for g in range(GROUPS):
            for e in range(16):
                row = g * 16 + e
                parts = []
                for j in range(D // 16):
                    vs = rs[row, pl.ds(j * 16, 16)]
                    vd = rd[row, pl.ds(j * 16, 16)]
                    parts.append(vs * vd)
                while len(parts) > 1:
                    parts = [a + b_ for a, b_ in
                             zip(parts[::2], parts[1::2])]
                plsc.store_scatter(trans_v, [lane16 + e], parts[0])
            colsums = [trans_v[pl.ds(l * 16, 16)] for l in range(16)]
            while len(colsums) > 1:
                colsums = [a + b_ for a, b_ in
                           zip(colsums[::2], colsums[1::2])]
            out_v[pl.ds(obase + g * 16, 16)] = colsums[0]

    start_gather(0, 0)

    def pair_body(k, _):
        c = 2 * k
        start_gather(c + 1, 1)
        wait_gather(c, 0)
        compute(c, 0)
        start_gather(c + 2, 0)
        wait_gather(c + 1, 1)
        compute(c + 1, 1)
        return ()

    lax.fori_loop(0, (NCHUNK - 1) // 2, pair_body, ())
    c_last = NCHUNK - 1
    wait_gather(c_last, 0)
    compute(c_last, 0)

    pltpu.sync_copy(out_v, out_hbm.at[pl.ds(base, EPW)])


@jax.jit
def _run(z_src, z_dst, src_idx, dst_idx):
    mesh = plsc.VectorSubcoreMesh(core_axis_name="c", subcore_axis_name="s")
    f = pl.kernel(
        _body,
        out_type=jax.ShapeDtypeStruct((E,), jnp.float32),
        mesh=mesh,
        compiler_params=pltpu.CompilerParams(needs_layout_passes=False),
        scratch_types=[
            pltpu.VMEM((EPW,), jnp.int32),
            pltpu.VMEM((EPW,), jnp.int32),
            pltpu.VMEM((CHUNK, D), jnp.float32),
            pltpu.VMEM((CHUNK, D), jnp.float32),
            pltpu.VMEM((CHUNK, D), jnp.float32),
            pltpu.VMEM((CHUNK, D), jnp.float32),
            pltpu.VMEM((EPW,), jnp.float32),
            pltpu.VMEM((256,), jnp.float32),
            pltpu.SemaphoreType.DMA,
            pltpu.SemaphoreType.DMA,
            pltpu.SemaphoreType.DMA,
            pltpu.SemaphoreType.DMA,
        ],
    )
    return f(z_src, z_dst, src_idx, dst_idx)


def kernel(z_src, z_dst, edge_label_index):
    return _run(z_src, z_dst, edge_label_index[0], edge_label_index[1])


# speedup vs baseline: 4.0708x; 1.1740x over previous
import jax
import jax.numpy as jnp
from jax import lax
from jax.experimental import pallas as pl
from jax.experimental.pallas import tpu as pltpu
from jax.experimental.pallas import tpu_sc as plsc

N_NODES = 10000
D = 128
E = 320000
NC = 2
NS = 16
NW = NC * NS
EPW = E // NW
CHUNK = 80
NCHUNK = EPW // CHUNK
GROUPS = CHUNK // 16


def _body(zs_hbm, zd_hbm, sidx_hbm, didx_hbm, out_hbm,
          sidx_v, didx_v, rows_s0, rows_d0, rows_s1, rows_d1, out_v,
          trans_v, sem_s0, sem_d0, sem_s1, sem_d1):
    wid = lax.axis_index("s") * NC + lax.axis_index("c")
    base = wid * EPW
    lane16 = lax.iota(jnp.int32, 16) * 16

    pltpu.sync_copy(sidx_hbm.at[pl.ds(base, EPW)], sidx_v)
    pltpu.sync_copy(didx_hbm.at[pl.ds(base, EPW)], didx_v)

    bufs = ((rows_s0, rows_d0, sem_s0, sem_d0),
            (rows_s1, rows_d1, sem_s1, sem_d1))

    def start_gather(c, b):
        rs, rd, ss, sd = bufs[b]
        pltpu.async_copy(zs_hbm.at[sidx_v.at[pl.ds(c * CHUNK, CHUNK)]], rs, ss)
        pltpu.async_copy(zd_hbm.at[didx_v.at[pl.ds(c * CHUNK, CHUNK)]], rd, sd)

    def wait_gather(c, b):
        rs, rd, ss, sd = bufs[b]
        pltpu.make_async_copy(
            zs_hbm.at[sidx_v.at[pl.ds(c * CHUNK, CHUNK)]], rs, ss).wait()
        pltpu.make_async_copy(
            zd_hbm.at[didx_v.at[pl.ds(c * CHUNK, CHUNK)]], rd, sd).wait()

    def compute(c, b):
        rs, rd, _, _ = bufs[b]
        obase = c * CHUNK
        for g in range(GROUPS):
            for e in range(16):
                row = g * 16 + e
                parts = []
                for j in range(D // 32):
                    vs = plsc.bitcast(rs[row, pl.ds(j * 16, 16)],
                                      jnp.bfloat16)
                    vd = plsc.bitcast(rd[row, pl.ds(j * 16, 16)],
                                      jnp.bfloat16)
                    p_lo, p_hi = plsc.unpack(
                        vs * vd, format=plsc.PackFormat.INTERLEAVED)
                    parts.append(p_lo)
                    parts.append(p_hi)
                while len(parts) > 1:
                    parts = [a + b_ for a, b_ in
                             zip(parts[::2], parts[1::2])]
                plsc.store_scatter(trans_v, [lane16 + e], parts[0])
            colsums = [trans_v[pl.ds(l * 16, 16)] for l in range(16)]
            while len(colsums) > 1:
                colsums = [a + b_ for a, b_ in
                           zip(colsums[::2], colsums[1::2])]
            out_v[pl.ds(obase + g * 16, 16)] = colsums[0]

    start_gather(0, 0)

    def pair_body(k, _):
        c = 2 * k
        start_gather(c + 1, 1)
        wait_gather(c, 0)
        compute(c, 0)
        start_gather(c + 2, 0)
        wait_gather(c + 1, 1)
        compute(c + 1, 1)
        return ()

    lax.fori_loop(0, (NCHUNK - 1) // 2, pair_body, ())
    c_last = NCHUNK - 1
    wait_gather(c_last, 0)
    compute(c_last, 0)

    pltpu.sync_copy(out_v, out_hbm.at[pl.ds(base, EPW)])


@jax.jit
def _run(z_src, z_dst, src_idx, dst_idx):
    mesh = plsc.VectorSubcoreMesh(core_axis_name="c", subcore_axis_name="s")
    f = pl.kernel(
        _body,
        out_type=jax.ShapeDtypeStruct((E,), jnp.float32),
        mesh=mesh,
        compiler_params=pltpu.CompilerParams(
            needs_layout_passes=False, use_tc_tiling_on_sc=False),
        scratch_types=[
            pltpu.VMEM((EPW,), jnp.int32),
            pltpu.VMEM((EPW,), jnp.int32),
            pltpu.VMEM((CHUNK, D // 2), jnp.int32),
            pltpu.VMEM((CHUNK, D // 2), jnp.int32),
            pltpu.VMEM((CHUNK, D // 2), jnp.int32),
            pltpu.VMEM((CHUNK, D // 2), jnp.int32),
            pltpu.VMEM((EPW,), jnp.float32),
            pltpu.VMEM((256,), jnp.float32),
            pltpu.SemaphoreType.DMA,
            pltpu.SemaphoreType.DMA,
            pltpu.SemaphoreType.DMA,
            pltpu.SemaphoreType.DMA,
        ],
    )
    return f(z_src, z_dst, src_idx, dst_idx)


def _pack_bf16(z):
    zb = z.astype(jnp.bfloat16).reshape(z.shape[0], z.shape[1] // 2, 2)
    return jax.lax.bitcast_convert_type(zb, jnp.int32)


def kernel(z_src, z_dst, edge_label_index):
    return _run(_pack_bf16(z_src), _pack_bf16(z_dst),
                edge_label_index[0], edge_label_index[1])
